# Initial kernel scaffold; baseline (speedup 1.0000x reference)
import jax
import jax.numpy as jnp
from jax.experimental import pallas as pl


def kernel(x, w1, b1, w2, b2, wh, bh):
    raise NotImplementedError("write your pallas kernel here")


# speedup vs baseline: 1.2654x; 1.2654x over previous
import functools

import jax
import jax.numpy as jnp
from jax.experimental import pallas as pl
from jax.experimental.pallas import tpu as pltpu

_OBS = 8
_ACT = 4
_HID = 64
_PACK = 16
_T16 = 512
_STEP = _PACK * _T16


def _ac_kernel(x_ref, w1_ref, b1_ref, w2_ref, b2_ref,
               wl_ref, bl_ref, wv_ref, bv_ref,
               logits_ref, value_ref):
    xb = x_ref[...]
    h = jnp.tanh(
        jnp.dot(xb, w1_ref[...], preferred_element_type=jnp.float32)
        + b1_ref[...]
    )
    h = jnp.tanh(
        jnp.dot(h, w2_ref[...], preferred_element_type=jnp.float32)
        + b2_ref[...]
    )
    lw = (jnp.dot(h, wl_ref[...], preferred_element_type=jnp.float32)
          + bl_ref[...])
    vw = (jnp.dot(h, wv_ref[...], preferred_element_type=jnp.float32)
          + bv_ref[...])
    logits_ref[...] = lw
    value_ref[...] = vw


@functools.partial(jax.jit, static_argnames=("rows",))
def _forward(xd, w1b, b1b, w2b, b2b, wlb, blb, wvb, bvb, *, rows):
    grid = (rows // _T16,)
    logits_d, value_d = pl.pallas_call(
        _ac_kernel,
        grid=grid,
        in_specs=[
            pl.BlockSpec((_T16, 128), lambda i: (i, 0)),
            pl.BlockSpec((128, 1024), lambda i: (0, 0)),
            pl.BlockSpec((1, 1024), lambda i: (0, 0)),
            pl.BlockSpec((1024, 1024), lambda i: (0, 0)),
            pl.BlockSpec((1, 1024), lambda i: (0, 0)),
            pl.BlockSpec((1024, 64), lambda i: (0, 0)),
            pl.BlockSpec((1, 64), lambda i: (0, 0)),
            pl.BlockSpec((1024, 16), lambda i: (0, 0)),
            pl.BlockSpec((1, 16), lambda i: (0, 0)),
        ],
        out_specs=[
            pl.BlockSpec((_T16, _PACK * _ACT), lambda i: (i, 0)),
            pl.BlockSpec((_T16, _PACK), lambda i: (i, 0)),
        ],
        out_shape=[
            jax.ShapeDtypeStruct((rows, _PACK * _ACT), jnp.float32),
            jax.ShapeDtypeStruct((rows, _PACK), jnp.float32),
        ],
        compiler_params=pltpu.CompilerParams(
            dimension_semantics=("parallel",),
        ),
    )(xd, w1b, b1b, w2b, b2b, wlb, blb, wvb, bvb)
    return logits_d, value_d


def kernel(x, w1, b1, w2, b2, wh, bh):
    B = x.shape[0]
    Bp = -(-B // _STEP) * _STEP
    if Bp != B:
        x = jnp.pad(x, ((0, Bp - B), (0, 0)))
    xd = x.reshape(Bp // _PACK, _PACK * _OBS)

    eye = jnp.eye(_PACK, dtype=jnp.float32)
    w1b = jnp.kron(eye, w1[:, :_HID])
    b1b = jnp.tile(b1[:, :_HID], (1, _PACK))
    w2b = jnp.kron(eye, w2[:_HID, :_HID])
    b2b = jnp.tile(b2[:, :_HID], (1, _PACK))
    wlb = jnp.kron(eye, wh[:_HID, :_ACT])
    blb = jnp.tile(bh[:, :_ACT], (1, _PACK))
    wvb = jnp.kron(eye, wh[:_HID, _ACT:_ACT + 1])
    bvb = jnp.tile(bh[:, _ACT:_ACT + 1], (1, _PACK))

    rows = Bp // _PACK
    logits_d, value_d = _forward(
        xd, w1b, b1b, w2b, b2b, wlb, blb, wvb, bvb, rows=rows)
    logits = logits_d.reshape(Bp, _ACT)[:B]
    value = value_d.reshape(Bp, 1)[:B]
    return logits, value


# speedup vs baseline: 1.7212x; 1.3602x over previous
import functools

import jax
import jax.numpy as jnp
from jax.experimental import pallas as pl
from jax.experimental.pallas import tpu as pltpu

_OBS = 8
_ACT = 4
_HID = 64
_TILE = 2048


def _ac_kernel(x_ref, w1a_ref, w1b_ref, b1_ref, w2_ref, b2_ref,
               wla_ref, wlb_ref, bl_ref, wva_ref, wvb_ref, bv_ref,
               logits_ref, value_ref):
    xa = x_ref[0]
    xb = x_ref[1]
    h1 = jnp.tanh(
        jnp.dot(xa, w1a_ref[...], preferred_element_type=jnp.float32)
        + jnp.dot(xb, w1b_ref[...], preferred_element_type=jnp.float32)
        + b1_ref[...]
    )
    h2 = jnp.tanh(
        jnp.dot(h1, w2_ref[...], preferred_element_type=jnp.float32)
        + b2_ref[...]
    )
    logits_ref[0] = (
        jnp.dot(h2, wla_ref[...], preferred_element_type=jnp.float32)
        + bl_ref[...]
    )
    logits_ref[1] = (
        jnp.dot(h2, wlb_ref[...], preferred_element_type=jnp.float32)
        + bl_ref[...]
    )
    value_ref[0] = (
        jnp.dot(h2, wva_ref[...], preferred_element_type=jnp.float32)
        + bv_ref[...]
    )
    value_ref[1] = (
        jnp.dot(h2, wvb_ref[...], preferred_element_type=jnp.float32)
        + bv_ref[...]
    )


@functools.partial(jax.jit, static_argnames=("half",))
def _forward(x3, w1a, w1b, b1p, w2p, b2p, wla, wlb, bl, wva, wvb, bv, *, half):
    grid = (half // _TILE,)
    logits3, value3 = pl.pallas_call(
        _ac_kernel,
        grid=grid,
        in_specs=[
            pl.BlockSpec((2, _TILE, _OBS), lambda i: (0, i, 0)),
            pl.BlockSpec((_OBS, 128), lambda i: (0, 0)),
            pl.BlockSpec((_OBS, 128), lambda i: (0, 0)),
            pl.BlockSpec((1, 128), lambda i: (0, 0)),
            pl.BlockSpec((128, 128), lambda i: (0, 0)),
            pl.BlockSpec((1, 128), lambda i: (0, 0)),
            pl.BlockSpec((128, _ACT), lambda i: (0, 0)),
            pl.BlockSpec((128, _ACT), lambda i: (0, 0)),
            pl.BlockSpec((1, _ACT), lambda i: (0, 0)),
            pl.BlockSpec((128, 1), lambda i: (0, 0)),
            pl.BlockSpec((128, 1), lambda i: (0, 0)),
            pl.BlockSpec((1, 1), lambda i: (0, 0)),
        ],
        out_specs=[
            pl.BlockSpec((2, _TILE, _ACT), lambda i: (0, i, 0)),
            pl.BlockSpec((2, _TILE, 1), lambda i: (0, i, 0)),
        ],
        out_shape=[
            jax.ShapeDtypeStruct((2, half, _ACT), jnp.float32),
            jax.ShapeDtypeStruct((2, half, 1), jnp.float32),
        ],
        compiler_params=pltpu.CompilerParams(
            dimension_semantics=("parallel",),
        ),
    )(x3, w1a, w1b, b1p, w2p, b2p, wla, wlb, bl, wva, wvb, bv)
    return logits3, value3


def kernel(x, w1, b1, w2, b2, wh, bh):
    B = x.shape[0]
    half = -(-B // (2 * _TILE)) * _TILE
    if 2 * half != B:
        x = jnp.pad(x, ((0, 2 * half - B), (0, 0)))
    x3 = x.reshape(2, half, _OBS)

    w1c = w1[:, :_HID]
    w1a = jnp.zeros((_OBS, 128), jnp.float32).at[:, :_HID].set(w1c)
    w1b = jnp.zeros((_OBS, 128), jnp.float32).at[:, _HID:].set(w1c)
    b1c = b1[:, :_HID]
    b1p = jnp.concatenate([b1c, b1c], axis=1)
    w2c = w2[:_HID, :_HID]
    w2p = (jnp.zeros((128, 128), jnp.float32)
           .at[:_HID, :_HID].set(w2c)
           .at[_HID:, _HID:].set(w2c))
    b2c = b2[:, :_HID]
    b2p = jnp.concatenate([b2c, b2c], axis=1)
    wa = wh[:_HID, :_ACT]
    wla = jnp.zeros((128, _ACT), jnp.float32).at[:_HID].set(wa)
    wlb = jnp.zeros((128, _ACT), jnp.float32).at[_HID:].set(wa)
    bl = bh[:, :_ACT]
    wc = wh[:_HID, _ACT:_ACT + 1]
    wva = jnp.zeros((128, 1), jnp.float32).at[:_HID].set(wc)
    wvb = jnp.zeros((128, 1), jnp.float32).at[_HID:].set(wc)
    bv = bh[:, _ACT:_ACT + 1]

    logits3, value3 = _forward(
        x3, w1a, w1b, b1p, w2p, b2p, wla, wlb, bl, wva, wvb, bv, half=half)
    logits = logits3.reshape(2 * half, _ACT)[:B]
    value = value3.reshape(2 * half, 1)[:B]
    return logits, value
